# baseline (speedup 1.0000x reference)
import functools

import jax
import jax.numpy as jnp
from jax import lax
from jax.experimental import pallas as pl
from jax.experimental.pallas import tpu as pltpu
from jax.experimental.pallas import tpu_sc as plsc

N_NODES = 100000
D = 128
B = 16384
F1 = 10
F2 = 5
NV = D // 16

PER_W = B // 32
TCH = 64
NT = PER_W // TCH
E1 = 32
E2 = 64

_MESH = plsc.VectorSubcoreMesh(core_axis_name="c", subcore_axis_name="s")
_INFO = plsc.get_sparse_core_info()


def _worker_id():
    return lax.axis_index("s") * _INFO.num_cores + lax.axis_index("c")


def _reduce_block(bufs, out_v, elems):
    def elem_body(i, carry):
        for u in range(2):
            e = 2 * i + u
            accs = [bufs[0][e, pl.ds(v * 16, 16)] for v in range(NV)]
            for j in range(1, len(bufs)):
                for v in range(NV):
                    accs[v] = accs[v] + bufs[j][e, pl.ds(v * 16, 16)]
            for v in range(NV):
                out_v[e, pl.ds(v * 16, 16)] = accs[v]
        return carry

    lax.fori_loop(0, elems // 2, elem_body, 0)


def _sc_gather_neighbors(feats, nids_T, elems, fanout, tids=None,
                         per_w=PER_W, off=0):
    nblk = per_w // elems
    npair = nblk // 2
    with_t = tids is not None

    out_type = [jax.ShapeDtypeStruct((B, D), jnp.float32)]
    scratch = [pltpu.VMEM((fanout, per_w), jnp.int32)]
    scratch += [pltpu.VMEM((elems, D), jnp.float32)
                for _ in range(2 * fanout)]
    scratch += [pltpu.VMEM((elems, D), jnp.float32)] * 2
    scratch += [pltpu.SemaphoreType.DMA] * 4
    if with_t:
        out_type.append(jax.ShapeDtypeStruct((B, D), jnp.float32))
        scratch += [
            pltpu.VMEM((PER_W,), jnp.int32),
            pltpu.VMEM((TCH, D), jnp.float32),
            pltpu.VMEM((TCH, D), jnp.float32),
            pltpu.SemaphoreType.DMA,
            pltpu.SemaphoreType.DMA,
            pltpu.SemaphoreType.DMA,
            pltpu.SemaphoreType.DMA,
        ]

    @functools.partial(
        pl.kernel, out_type=out_type, mesh=_MESH, scratch_types=scratch
    )
    def sc_kernel(feats_hbm, nid_hbm, *args):
        args = list(args)
        tids_hbm = args.pop(0) if with_t else None
        out_hbm = args.pop(0)
        ht_hbm = args.pop(0) if with_t else None
        idx_v = args.pop(0)
        setA = [args.pop(0) for _ in range(fanout)]
        setB = [args.pop(0) for _ in range(fanout)]
        outA, outB = args.pop(0), args.pop(0)
        semA, semB, soA, soB = (args.pop(0) for _ in range(4))
        if with_t:
            (idxT, rT0, rT1, tg0, tg1, to0, to1) = args

        wid = _worker_id()
        wbase = off + wid * per_w

        def fire_set(bufs, sem, blk):
            for j in range(fanout):
                pltpu.async_copy(
                    feats_hbm.at[idx_v.at[j, pl.ds(blk * elems, elems)]],
                    bufs[j], sem,
                )

        def wait_set(bufs, sem, blk):
            for j in range(fanout):
                pltpu.make_async_copy(
                    feats_hbm.at[idx_v.at[j, pl.ds(blk * elems, elems)]],
                    bufs[j], sem,
                ).wait()

        def fire_out(outb, sem, blk):
            pltpu.async_copy(
                outb, out_hbm.at[pl.ds(wbase + blk * elems, elems)], sem
            )

        def wait_out(outb, sem, blk):
            pltpu.make_async_copy(
                outb, out_hbm.at[pl.ds(wbase + blk * elems, elems)], sem
            ).wait()

        if with_t:
            rT = (rT0, rT1)
            tg = (tg0, tg1)
            to = (to0, to1)

            def t_gather(c, half):
                pltpu.async_copy(
                    feats_hbm.at[idxT.at[pl.ds(c * TCH, TCH)]],
                    rT[half], tg[half],
                )

            def t_wait_g(c, half):
                pltpu.make_async_copy(
                    feats_hbm.at[idxT.at[pl.ds(c * TCH, TCH)]],
                    rT[half], tg[half],
                ).wait()

            def t_out(c, half):
                pltpu.async_copy(
                    rT[half], ht_hbm.at[pl.ds(wbase + c * TCH, TCH)], to[half]
                )

            def t_wait_o(c, half):
                pltpu.make_async_copy(
                    rT[half], ht_hbm.at[pl.ds(wbase + c * TCH, TCH)], to[half]
                ).wait()

        pltpu.sync_copy(
            nid_hbm.at[pl.ds(0, fanout), pl.ds(wbase, per_w)], idx_v
        )
        if with_t:
            pltpu.sync_copy(tids_hbm.at[pl.ds(wbase, per_w)], idxT)
            t_gather(0, 0)
            t_gather(1, 1)
        fire_set(setA, semA, 0)
        fire_set(setB, semB, 1)

        def body(t, carry):
            if with_t:
                for k in range(1, NT):
                    @pl.when(t == k)
                    def _(k=k):
                        if k % 2 == 1:
                            t_wait_g(k - 1, 0); t_out(k - 1, 0)
                            t_wait_g(k, 1); t_out(k, 1)
                        else:
                            t_wait_o(k - 2, 0); t_gather(k, 0)
                            t_wait_o(k - 1, 1); t_gather(k + 1, 1)

            for parity, (bufs, sem, outb, so) in enumerate(
                ((setA, semA, outA, soA), (setB, semB, outB, soB))
            ):
                blk = 2 * t + parity
                wait_set(bufs, sem, blk)

                @pl.when(t >= 1)
                def _():
                    wait_out(outb, so, blk)

                _reduce_block(bufs, outb, elems)

                @pl.when(t + 1 < npair)
                def _():
                    fire_set(bufs, sem, blk + 2)

                fire_out(outb, so, blk)
            return carry

        lax.fori_loop(0, npair, body, 0)
        wait_out(outA, soA, 2 * npair - 2)
        wait_out(outB, soB, 2 * npair - 1)
        if with_t:
            t_wait_o(NT - 2, 0)
            t_wait_o(NT - 1, 1)

    if with_t:
        return sc_kernel(feats, nids_T, tids)
    return sc_kernel(feats, nids_T)[0]


def _tc_dense_layer(x, s, Wa, Wb, b, inv_n):
    BLK = 2048
    grid = (B // BLK,)
    dx = x.shape[1]

    def body(x_r, s_r, wa_r, wb_r, b_r, out_r):
        acc = (
            jnp.dot(x_r[...], wa_r[...], preferred_element_type=jnp.float32)
            + jnp.dot(s_r[...] * inv_n, wb_r[...],
                      preferred_element_type=jnp.float32)
            + b_r[...]
        )
        out_r[...] = jnp.maximum(acc, 0.0)

    full = lambda shape: pl.BlockSpec(shape, lambda i: tuple(0 for _ in shape))
    return pl.pallas_call(
        body,
        grid=grid,
        in_specs=[
            pl.BlockSpec((BLK, dx), lambda i: (i, 0)),
            pl.BlockSpec((BLK, D), lambda i: (i, 0)),
            full((dx, 64)), full((D, 64)), full((1, 64)),
        ],
        out_specs=pl.BlockSpec((BLK, 64), lambda i: (i, 0)),
        out_shape=jax.ShapeDtypeStruct((B, 64), jnp.float32),
    )(x, s, Wa, Wb, b)


def _tc_dense_layer_T(x, s, Wa, Wb, b_col, inv_n, half, prev=None):
    HB = B // 2
    dx = x.shape[1]

    def body(*refs):
        x_r, s_r, wa_r, wb_r, b_r, out_r = refs[-6:]
        acc = (
            lax.dot_general(wa_r[...], x_r[...], (((0,), (1,)), ((), ())),
                            preferred_element_type=jnp.float32)
            + lax.dot_general(wb_r[...], s_r[...] * inv_n,
                              (((0,), (1,)), ((), ())),
                              preferred_element_type=jnp.float32)
            + b_r[...]
        )
        out_r[...] = jnp.maximum(acc, 0.0)

    full = lambda shape: pl.BlockSpec(shape, lambda i: tuple(0 for _ in shape))
    in_specs = [
        pl.BlockSpec((HB, dx), lambda i: (half, 0)),
        pl.BlockSpec((HB, D), lambda i: (half, 0)),
        full((dx, 64)), full((D, 64)), full((64, 1)),
    ]
    args = (x, s, Wa, Wb, b_col)
    aliases = {}
    if prev is not None:
        in_specs = [pl.BlockSpec(memory_space=pltpu.MemorySpace.HBM)] + in_specs
        args = (prev,) + args
        aliases = {0: 0}
    return pl.pallas_call(
        body,
        grid=(1,),
        in_specs=in_specs,
        out_specs=pl.BlockSpec((64, HB), lambda i: (0, half)),
        out_shape=jax.ShapeDtypeStruct((64, B), jnp.float32),
        input_output_aliases=aliases,
    )(*args)


def kernel(target_node_ids, all_node_features, neighbor_ids_l1, neighbor_ids_l2,
           W1, b1, W2, b2):
    tids = target_node_ids.astype(jnp.int32)
    n1_T = neighbor_ids_l1.astype(jnp.int32).T
    n2_T = neighbor_ids_l2.astype(jnp.int32).T
    s1, ht = _sc_gather_neighbors(all_node_features, n1_T, E1, F1, tids=tids)
    HW = PER_W // 2
    s2a = _sc_gather_neighbors(all_node_features, n2_T, E2, F2,
                               per_w=HW, off=0)
    s2b = _sc_gather_neighbors(all_node_features, n2_T, E2, F2,
                               per_w=HW, off=B // 2)
    h1 = _tc_dense_layer(ht, s1, W1[:D], W1[D:], b1.reshape(1, -1), 1.0 / F1)
    b2c = b2.reshape(-1, 1)
    h2T_a = _tc_dense_layer_T(h1, s2a, W2[:64], W2[64:], b2c, 1.0 / F2, 0)
    h2T = _tc_dense_layer_T(h1, s2b, W2[:64], W2[64:], b2c, 1.0 / F2, 1,
                            prev=h2T_a)
    return h2T.T


# speedup vs baseline: 1.0764x; 1.0764x over previous
import functools

import jax
import jax.numpy as jnp
from jax import lax
from jax.experimental import pallas as pl
from jax.experimental.pallas import tpu as pltpu
from jax.experimental.pallas import tpu_sc as plsc

N_NODES = 100000
D = 128
B = 16384
F1 = 10
F2 = 5
NV = D // 16

PER_W = B // 32
TCH = 64
NT = PER_W // TCH
E1 = 32
E2 = 64

_MESH = plsc.VectorSubcoreMesh(core_axis_name="c", subcore_axis_name="s")
_INFO = plsc.get_sparse_core_info()


def _worker_id():
    return lax.axis_index("s") * _INFO.num_cores + lax.axis_index("c")


def _reduce_block(bufs, out_v, elems):
    def elem_body(i, carry):
        for u in range(2):
            e = 2 * i + u
            accs = [bufs[0][e, pl.ds(v * 16, 16)] for v in range(NV)]
            for j in range(1, len(bufs)):
                for v in range(NV):
                    accs[v] = accs[v] + bufs[j][e, pl.ds(v * 16, 16)]
            for v in range(NV):
                out_v[e, pl.ds(v * 16, 16)] = accs[v]
        return carry

    lax.fori_loop(0, elems // 2, elem_body, 0)


def _sc_gather_neighbors(feats, nids_T, elems, fanout, tids=None):
    nblk = PER_W // elems
    npair = nblk // 2
    with_t = tids is not None

    out_type = [jax.ShapeDtypeStruct((B, D), jnp.float32)]
    scratch = [pltpu.VMEM((fanout, PER_W), jnp.int32)]
    scratch += [pltpu.VMEM((elems, D), jnp.float32)
                for _ in range(2 * fanout)]
    scratch += [pltpu.VMEM((elems, D), jnp.float32)] * 2
    scratch += [pltpu.SemaphoreType.DMA] * 4
    if with_t:
        out_type.append(jax.ShapeDtypeStruct((B, D), jnp.float32))
        scratch += [
            pltpu.VMEM((PER_W,), jnp.int32),
            pltpu.VMEM((TCH, D), jnp.float32),
            pltpu.VMEM((TCH, D), jnp.float32),
            pltpu.SemaphoreType.DMA,
            pltpu.SemaphoreType.DMA,
            pltpu.SemaphoreType.DMA,
            pltpu.SemaphoreType.DMA,
        ]

    @functools.partial(
        pl.kernel, out_type=out_type, mesh=_MESH, scratch_types=scratch
    )
    def sc_kernel(feats_hbm, nid_hbm, *args):
        args = list(args)
        tids_hbm = args.pop(0) if with_t else None
        out_hbm = args.pop(0)
        ht_hbm = args.pop(0) if with_t else None
        idx_v = args.pop(0)
        setA = [args.pop(0) for _ in range(fanout)]
        setB = [args.pop(0) for _ in range(fanout)]
        outA, outB = args.pop(0), args.pop(0)
        semA, semB, soA, soB = (args.pop(0) for _ in range(4))
        if with_t:
            (idxT, rT0, rT1, tg0, tg1, to0, to1) = args

        wid = _worker_id()
        wbase = wid * PER_W

        def fire_set(bufs, sem, blk):
            for j in range(fanout):
                pltpu.async_copy(
                    feats_hbm.at[idx_v.at[j, pl.ds(blk * elems, elems)]],
                    bufs[j], sem,
                )

        def wait_set(bufs, sem, blk):
            for j in range(fanout):
                pltpu.make_async_copy(
                    feats_hbm.at[idx_v.at[j, pl.ds(blk * elems, elems)]],
                    bufs[j], sem,
                ).wait()

        def fire_out(outb, sem, blk):
            pltpu.async_copy(
                outb, out_hbm.at[pl.ds(wbase + blk * elems, elems)], sem
            )

        def wait_out(outb, sem, blk):
            pltpu.make_async_copy(
                outb, out_hbm.at[pl.ds(wbase + blk * elems, elems)], sem
            ).wait()

        if with_t:
            rT = (rT0, rT1)
            tg = (tg0, tg1)
            to = (to0, to1)

            def t_gather(c, half):
                pltpu.async_copy(
                    feats_hbm.at[idxT.at[pl.ds(c * TCH, TCH)]],
                    rT[half], tg[half],
                )

            def t_wait_g(c, half):
                pltpu.make_async_copy(
                    feats_hbm.at[idxT.at[pl.ds(c * TCH, TCH)]],
                    rT[half], tg[half],
                ).wait()

            def t_out(c, half):
                pltpu.async_copy(
                    rT[half], ht_hbm.at[pl.ds(wbase + c * TCH, TCH)], to[half]
                )

            def t_wait_o(c, half):
                pltpu.make_async_copy(
                    rT[half], ht_hbm.at[pl.ds(wbase + c * TCH, TCH)], to[half]
                ).wait()

        pltpu.sync_copy(
            nid_hbm.at[pl.ds(0, fanout), pl.ds(wbase, PER_W)], idx_v
        )
        if with_t:
            pltpu.sync_copy(tids_hbm.at[pl.ds(wbase, PER_W)], idxT)
            t_gather(0, 0)
            t_gather(1, 1)
        fire_set(setA, semA, 0)
        fire_set(setB, semB, 1)

        def body(t, carry):
            if with_t:
                for k in range(1, NT):
                    @pl.when(t == k)
                    def _(k=k):
                        if k % 2 == 1:
                            t_wait_g(k - 1, 0); t_out(k - 1, 0)
                            t_wait_g(k, 1); t_out(k, 1)
                        else:
                            t_wait_o(k - 2, 0); t_gather(k, 0)
                            t_wait_o(k - 1, 1); t_gather(k + 1, 1)

            for parity, (bufs, sem, outb, so) in enumerate(
                ((setA, semA, outA, soA), (setB, semB, outB, soB))
            ):
                blk = 2 * t + parity
                wait_set(bufs, sem, blk)

                @pl.when(t >= 1)
                def _():
                    wait_out(outb, so, blk)

                _reduce_block(bufs, outb, elems)

                @pl.when(t + 1 < npair)
                def _():
                    fire_set(bufs, sem, blk + 2)

                fire_out(outb, so, blk)
            return carry

        lax.fori_loop(0, npair, body, 0)
        wait_out(outA, soA, 2 * npair - 2)
        wait_out(outB, soB, 2 * npair - 1)
        if with_t:
            t_wait_o(NT - 2, 0)
            t_wait_o(NT - 1, 1)

    if with_t:
        return sc_kernel(feats, nids_T, tids)
    return sc_kernel(feats, nids_T)[0]


def _tc_dense_layer(x, s, Wa, Wb, b, inv_n):
    BLK = 2048
    grid = (B // BLK,)
    dx = x.shape[1]

    def body(x_r, s_r, wa_r, wb_r, b_r, out_r):
        acc = (
            jnp.dot(x_r[...], wa_r[...], preferred_element_type=jnp.float32)
            + jnp.dot(s_r[...] * inv_n, wb_r[...],
                      preferred_element_type=jnp.float32)
            + b_r[...]
        )
        out_r[...] = jnp.maximum(acc, 0.0)

    full = lambda shape: pl.BlockSpec(shape, lambda i: tuple(0 for _ in shape))
    return pl.pallas_call(
        body,
        grid=grid,
        in_specs=[
            pl.BlockSpec((BLK, dx), lambda i: (i, 0)),
            pl.BlockSpec((BLK, D), lambda i: (i, 0)),
            full((dx, 64)), full((D, 64)), full((1, 64)),
        ],
        out_specs=pl.BlockSpec((BLK, 64), lambda i: (i, 0)),
        out_shape=jax.ShapeDtypeStruct((B, 64), jnp.float32),
    )(x, s, Wa, Wb, b)


def _tc_dense_layer_T(x, s, Wa, Wb, b_col, inv_n):
    BLK = 8192
    grid = (B // BLK,)
    dx = x.shape[1]

    def body(x_r, s_r, wa_r, wb_r, b_r, out_r):
        acc = (
            lax.dot_general(wa_r[...], x_r[...], (((0,), (1,)), ((), ())),
                            preferred_element_type=jnp.float32)
            + lax.dot_general(wb_r[...], s_r[...] * inv_n,
                              (((0,), (1,)), ((), ())),
                              preferred_element_type=jnp.float32)
            + b_r[...]
        )
        out_r[...] = jnp.maximum(acc, 0.0)

    full = lambda shape: pl.BlockSpec(shape, lambda i: tuple(0 for _ in shape))
    return pl.pallas_call(
        body,
        grid=grid,
        in_specs=[
            pl.BlockSpec((BLK, dx), lambda i: (i, 0)),
            pl.BlockSpec((BLK, D), lambda i: (i, 0)),
            full((dx, 64)), full((D, 64)), full((64, 1)),
        ],
        out_specs=pl.BlockSpec((64, BLK), lambda i: (0, i)),
        out_shape=jax.ShapeDtypeStruct((64, B), jnp.float32),
    )(x, s, Wa, Wb, b_col)


def kernel(target_node_ids, all_node_features, neighbor_ids_l1, neighbor_ids_l2,
           W1, b1, W2, b2):
    tids = target_node_ids.astype(jnp.int32)
    n1_T = neighbor_ids_l1.astype(jnp.int32).T
    n2_T = neighbor_ids_l2.astype(jnp.int32).T
    s1, ht = _sc_gather_neighbors(all_node_features, n1_T, E1, F1, tids=tids)
    s2 = _sc_gather_neighbors(all_node_features, n2_T, E2, F2)
    h1 = _tc_dense_layer(ht, s1, W1[:D], W1[D:], b1.reshape(1, -1), 1.0 / F1)
    h2T = _tc_dense_layer_T(h1, s2, W2[:64], W2[64:], b2.reshape(-1, 1),
                            1.0 / F2)
    return h2T.T
